# Initial kernel scaffold; baseline (speedup 1.0000x reference)
import jax
import jax.numpy as jnp
from jax.experimental import pallas as pl


def kernel(query, q_sub, q_rel, hidden, edges, nodes, Ws_attn_w, Wr_attn_w, Wqr_attn_w, Wqr_attn_b, W_attn_w, W_h_w, rela_embed, agg_w, agg_b, agg_ln_g, agg_ln_b, mlp_w0, mlp_b0, mlp_ln0_g, mlp_ln0_b, mlp_w1, mlp_b1, mlp_ln1_g, mlp_ln1_b):
    raise NotImplementedError("write your pallas kernel here")


# speedup vs baseline: 1.4581x; 1.4581x over previous
import functools

import jax
import jax.numpy as jnp
from jax import lax
from jax.experimental import pallas as pl
from jax.experimental.pallas import tpu as pltpu
from jax.experimental.pallas import tpu_sc as plsc

B, N, E = 2, 10000, 320000
BN = B * N
D, A, R = 128, 32, 400
H = 64
HP = H // 2
RW = HP + A
NC, NS = 2, 16
EPT = E // NS
FB = 400
NFB = EPT // FB
K = 80
CPB = FB // K
G16 = K // 16
BNP = 20480
RPT = BNP // NS


def _prep_body(ent_ref, rela_ref, query_ref, Ws_ref, Wr_ref, Wqr_ref, b_ref,
               hs_ref, hr_ref, hq_ref):
    ent = ent_ref[...]
    hs_ref[...] = lax.dot_general(ent, Ws_ref[...], (((1,), (1,)), ((), ())),
                                  preferred_element_type=jnp.float32)

    @pl.when(pl.program_id(0) == 0)
    def _():
        hr_ref[...] = lax.dot_general(rela_ref[...], Wr_ref[...],
                                      (((1,), (1,)), ((), ())),
                                      preferred_element_type=jnp.float32)
        hq_ref[...] = lax.dot_general(query_ref[...], Wqr_ref[...],
                                      (((1,), (1,)), ((), ())),
                                      preferred_element_type=jnp.float32) + b_ref[...]


def _run_prep(ent, rela, query8, Ws, Wr, Wqr, b):
    blk = 1000
    grid = BN // blk
    return pl.pallas_call(
        _prep_body,
        grid=(grid,),
        in_specs=[
            pl.BlockSpec((blk, D), lambda i: (i, 0)),
            pl.BlockSpec((R, D), lambda i: (0, 0)),
            pl.BlockSpec((8, D), lambda i: (0, 0)),
            pl.BlockSpec((A, D), lambda i: (0, 0)),
            pl.BlockSpec((A, D), lambda i: (0, 0)),
            pl.BlockSpec((A, D), lambda i: (0, 0)),
            pl.BlockSpec((1, A), lambda i: (0, 0)),
        ],
        out_specs=[
            pl.BlockSpec((blk, A), lambda i: (i, 0)),
            pl.BlockSpec((R, A), lambda i: (0, 0)),
            pl.BlockSpec((8, A), lambda i: (0, 0)),
        ],
        out_shape=[
            jax.ShapeDtypeStruct((BN, A), jnp.float32),
            jax.ShapeDtypeStruct((R, A), jnp.float32),
            jax.ShapeDtypeStruct((8, A), jnp.float32),
        ],
    )(ent, rela, query8, Ws, Wr, Wqr, b)


def _pack_halves(x):
    m = x.shape[0]
    u = lax.bitcast_convert_type(x.astype(jnp.bfloat16), jnp.uint16)
    u = u.astype(jnp.uint32).reshape(m, 2, HP, 2).transpose(1, 0, 2, 3)
    w = u[..., 0] | (u[..., 1] << 16)
    return lax.bitcast_convert_type(w, jnp.float32)


def _sc_body(tg, rg_hbm, hq_hbm, w_hbm, sub_hbm, rel_hbm, bat_hbm, obj_hbm,
             out_hbm,
             agg, rg_v, hq_v, w_v, sub_b, rel_b, bat_b, obj_b, G, msg, sem):
    cid = lax.axis_index("c")
    sid = lax.axis_index("s")
    ioe = lax.iota(jnp.int32, 16)

    pltpu.sync_copy(rg_hbm.at[cid], rg_v)
    pltpu.sync_copy(hq_hbm, hq_v)
    pltpu.sync_copy(w_hbm, w_v)
    wvec0 = w_v[pl.ds(0, 16)]
    wvec1 = w_v[pl.ds(16, 16)]

    zero16 = jnp.zeros((16,), jnp.float32)

    def _zb(i, c):
        msg[i // 4, pl.ds((i % 4) * 16, 16)] = zero16
        return c

    lax.fori_loop(0, K * (H // 16), _zb, 0)

    def _zcp(i, c):
        pltpu.sync_copy(msg, agg.at[pl.ds(sid * RPT + i * K, K)])
        return c

    lax.fori_loop(0, RPT // K, _zcp, 0)
    plsc.subcore_barrier()

    sub_off = cid * BN
    mask_hi = jnp.full((16,), 0xFFFF0000, jnp.uint32).astype(jnp.int32)

    def unpack2(word):
        wi = plsc.bitcast(word, jnp.int32)
        lo = plsc.bitcast(lax.shift_left(wi, 16), jnp.float32)
        hi = plsc.bitcast(jnp.bitwise_and(wi, mask_hi), jnp.float32)
        return lo, hi

    def fb_body(f, carry):
        pltpu.sync_copy(sub_hbm.at[sid, f], sub_b)
        pltpu.sync_copy(rel_hbm.at[sid, f], rel_b)
        pltpu.sync_copy(bat_hbm.at[sid, f], bat_b)
        pltpu.sync_copy(obj_hbm.at[sid, f], obj_b)

        def _addo(i, c):
            sub_b[pl.ds(i * 16, 16)] = sub_b[pl.ds(i * 16, 16)] + sub_off
            return c

        lax.fori_loop(0, FB // 16, _addo, 0)

        def chunk_body(cc, c2):
            pltpu.async_copy(tg.at[sub_b.at[pl.ds(cc * K, K)]], G, sem).wait()

            def g_body(g, c3):
                lrow = g * 16 + ioe
                relv = rel_b[pl.ds(cc * K + g * 16, 16)]
                batv = bat_b[pl.ds(cc * K + g * 16, 16)]
                acc = jnp.zeros((16,), jnp.float32)
                for j in range(A):
                    cj = jnp.full((16,), HP + j, jnp.int32)
                    a1 = plsc.load_gather(G, [lrow, cj])
                    a2 = plsc.load_gather(rg_v, [relv, cj])
                    a3 = plsc.load_gather(hq_v, [batv, jnp.full((16,), j, jnp.int32)])
                    t = jnp.maximum(a1 + a2 + a3, 0.0)
                    wj = wvec0[j] if j < 16 else wvec1[j - 16]
                    acc = acc + t * wj
                alpha = 1.0 / (1.0 + jnp.exp(-acc))
                for j in range(HP):
                    cj = jnp.full((16,), j, jnp.int32)
                    h0, h1 = unpack2(plsc.load_gather(G, [lrow, cj]))
                    r0, r1 = unpack2(plsc.load_gather(rg_v, [relv, cj]))
                    plsc.store_scatter(msg, [lrow, jnp.full((16,), 2 * j, jnp.int32)],
                                       h0 * r0 * alpha)
                    plsc.store_scatter(msg, [lrow, jnp.full((16,), 2 * j + 1, jnp.int32)],
                                       h1 * r1 * alpha)
                return c3

            lax.fori_loop(0, G16, g_body, 0)
            pltpu.sync_copy(msg, agg.at[obj_b.at[cc]], add=True)
            return c2

        lax.fori_loop(0, CPB, chunk_body, 0)
        return carry

    lax.fori_loop(0, NFB, fb_body, 0)
    plsc.subcore_barrier()
    pltpu.sync_copy(agg.at[pl.ds(sid * RPT, RPT)], out_hbm.at[cid, sid])


def _run_sc(tg, rg, hq, w, sub, rel, bat, obj):
    mesh = plsc.VectorSubcoreMesh(core_axis_name="c", subcore_axis_name="s")
    fn = functools.partial(
        pl.kernel,
        out_type=jax.ShapeDtypeStruct((NC, NS, RPT, H), jnp.float32),
        mesh=mesh,
        compiler_params=pltpu.CompilerParams(use_tc_tiling_on_sc=False,
                                             needs_layout_passes=False),
        scratch_types=[
            pltpu.VMEM_SHARED((BNP, H), jnp.float32),
            pltpu.VMEM((R, RW), jnp.float32),
            pltpu.VMEM((8, A), jnp.float32),
            pltpu.VMEM((A,), jnp.float32),
            pltpu.VMEM((FB,), jnp.int32),
            pltpu.VMEM((FB,), jnp.int32),
            pltpu.VMEM((FB,), jnp.int32),
            pltpu.VMEM((CPB, K), jnp.int32),
            pltpu.VMEM((K, RW), jnp.float32),
            pltpu.VMEM((K, H), jnp.float32),
            pltpu.SemaphoreType.DMA,
        ],
    )(_sc_body)
    return fn(tg, rg, hq, w, sub, rel, bat, obj)


def _post_body(a0_ref, a1_ref, w0a_ref, w0b_ref, ab_ref, ag_ref, abeta_ref,
               m0_ref, b0_ref, g0_ref, be0_ref,
               m1_ref, b1_ref, g1_ref, be1_ref, wh_ref, out_ref):
    def ln(x, g, b):
        m = jnp.mean(x, axis=-1, keepdims=True)
        v = jnp.mean((x - m) ** 2, axis=-1, keepdims=True)
        return (x - m) / jnp.sqrt(v + 1e-5) * g + b

    def mm(x, wt):
        return lax.dot_general(x, wt, (((1,), (0,)), ((), ())),
                               preferred_element_type=jnp.float32)

    z = mm(a0_ref[...], w0a_ref[...]) + mm(a1_ref[...], w0b_ref[...]) + ab_ref[...]
    z = ln(z, ag_ref[...], abeta_ref[...])
    h = jnp.maximum(ln(mm(z, m0_ref[...]) + b0_ref[...], g0_ref[...], be0_ref[...]), 0.0)
    h = jnp.maximum(ln(mm(h, m1_ref[...]) + b1_ref[...], g1_ref[...], be1_ref[...]), 0.0)
    out_ref[...] = jnp.maximum(mm(h, wh_ref[...]), 0.0)


def _run_post(a0, a1, w0at, w0bt, ab, ag, abeta,
              m0t, b0, g0, be0, m1t, b1, g1, be1, wht):
    blk = 1000
    grid = BN // blk
    row = lambda i: (i, 0)
    full = lambda i: (0, 0)
    wspec = pl.BlockSpec((D, D), full)
    vspec = pl.BlockSpec((1, D), full)
    return pl.pallas_call(
        _post_body,
        grid=(grid,),
        in_specs=[
            pl.BlockSpec((blk, H), row),
            pl.BlockSpec((blk, H), row),
            pl.BlockSpec((H, D), full),
            pl.BlockSpec((H, D), full),
            vspec, vspec, vspec,
            wspec, vspec, vspec, vspec,
            wspec, vspec, vspec, vspec,
            wspec,
        ],
        out_specs=pl.BlockSpec((blk, D), row),
        out_shape=jax.ShapeDtypeStruct((BN, D), jnp.float32),
    )(a0, a1, w0at, w0bt, ab, ag, abeta, m0t, b0, g0, be0, m1t, b1, g1, be1, wht)


def kernel(query, q_sub, q_rel, hidden, edges, nodes,
           Ws_attn_w, Wr_attn_w, Wqr_attn_w, Wqr_attn_b, W_attn_w,
           W_h_w, rela_embed,
           agg_w, agg_b, agg_ln_g, agg_ln_b,
           mlp_w0, mlp_b0, mlp_ln0_g, mlp_ln0_b,
           mlp_w1, mlp_b1, mlp_ln1_g, mlp_ln1_b):
    ent = hidden.reshape(BN, D)
    query8 = jnp.zeros((8, D), jnp.float32).at[:B].set(query)
    hs, hr, hq = _run_prep(ent, rela_embed, query8,
                           Ws_attn_w, Wr_attn_w, Wqr_attn_w,
                           Wqr_attn_b.reshape(1, A))

    entp = _pack_halves(ent)
    tg = jnp.concatenate(
        [entp, jnp.broadcast_to(hs[None], (NC, BN, A))], axis=-1)
    tg = tg.reshape(NC * BN, RW)
    relp = _pack_halves(rela_embed)
    rg = jnp.concatenate(
        [relp, jnp.broadcast_to(hr[None], (NC, R, A))], axis=-1)

    bat = edges[:, 0].reshape(NS, NFB, FB)
    sub = edges[:, 1].reshape(NS, NFB, FB)
    rel = edges[:, 2].reshape(NS, NFB, FB)
    obj = edges[:, 3].reshape(NS, NFB, CPB, K)

    agg4 = _run_sc(tg, rg, hq, W_attn_w[0], sub, rel, bat, obj)
    agg2 = agg4.reshape(NC, BNP, H)[:, :BN, :]

    out = _run_post(
        agg2[0], agg2[1],
        agg_w[:, :H].T, agg_w[:, H:].T, agg_b.reshape(1, D),
        agg_ln_g.reshape(1, D), agg_ln_b.reshape(1, D),
        mlp_w0.T, mlp_b0.reshape(1, D), mlp_ln0_g.reshape(1, D), mlp_ln0_b.reshape(1, D),
        mlp_w1.T, mlp_b1.reshape(1, D), mlp_ln1_g.reshape(1, D), mlp_ln1_b.reshape(1, D),
        W_h_w.T)
    return out.reshape(B, N, D)
